# Initial kernel scaffold; baseline (speedup 1.0000x reference)
import jax
import jax.numpy as jnp
from jax.experimental import pallas as pl


def kernel(x, edge_index, Wq0, bq0, Wk0, bk0, Wv0, bv0, Ws0, bs0, Wq1, bq1, Wk1, bk1, Wv1, bv1, Ws1, bs1, Wq2, bq2, Wk2, bk2, Wv2, bv2, Ws2, bs2):
    raise NotImplementedError("write your pallas kernel here")


# speedup vs baseline: 4.7092x; 4.7092x over previous
import functools
import math

import jax
import jax.numpy as jnp
from jax import lax
from jax.experimental import pallas as pl
from jax.experimental.pallas import tpu as pltpu
from jax.experimental.pallas import tpu_sc as plsc

N = 10000
E = 320000
D = 128
NC = 2
NS = 16
L = 16
NW = NC * NS
NP = 10240
NPT = NP // NW
EPTS = 10800
CHUNK = 80
NCH = EPTS // CHUNK
GPC = CHUNK // L
SCALE = 1.0 / math.sqrt(float(D))
BLK = 1000
NEG = -3.0e38


def _lane_sum(v, lanes):
    for sh in (8, 4, 2, 1):
        v = v + v.at[lanes ^ sh].get(mode="promise_in_bounds")
    return v


def _lane_max(v, lanes):
    for sh in (8, 4, 2, 1):
        v = jnp.maximum(v, v.at[lanes ^ sh].get(mode="promise_in_bounds"))
    return v


def _edge_body(q_hbm, k_hbm, v_hbm, srcp_hbm, dstp_hbm, meta_hbm,
               zagg_hbm, zden_hbm, agg_hbm, den_hbm,
               src1, dst1, alph, qrows, krows, metav, outb, denb,
               sem1, sem2):
    c = lax.axis_index("c")
    s = lax.axis_index("s")
    wid = c * NS + s
    node0 = wid * NPT
    node1 = node0 + NPT

    pltpu.sync_copy(meta_hbm.at[wid], metav)
    mv = metav[pl.ds(0, L)]
    st = mv[0]
    en = mv[1]
    st8 = (st // 8) * 8

    pltpu.sync_copy(srcp_hbm.at[pl.ds(st8, EPTS)], src1)
    pltpu.sync_copy(dstp_hbm.at[pl.ds(st8, EPTS)], dst1)
    pltpu.sync_copy(zagg_hbm, outb.at[pl.ds(0, NPT * D)])
    pltpu.sync_copy(zden_hbm, denb.at[pl.ds(0, NPT * L)])

    lanes = lax.iota(jnp.int32, L)

    def chunk_a(ch, mrun):
        qcp = pltpu.async_copy(q_hbm.at[dst1.at[pl.ds(ch * CHUNK, CHUNK)]],
                               qrows, sem1)
        kcp = pltpu.async_copy(k_hbm.at[src1.at[pl.ds(ch * CHUNK, CHUNK)]],
                               krows, sem2)
        qcp.wait()
        kcp.wait()

        def group(g, mrun):
            a16 = jnp.zeros((L,), jnp.float32)
            for u in range(L):
                e = g * L + u
                acc = jnp.zeros((L,), jnp.float32)
                for j in range(D // L):
                    acc = acc + (qrows[e, pl.ds(j * L, L)] *
                                 krows[e, pl.ds(j * L, L)])
                a16 = jnp.where(lanes == u, _lane_sum(acc, lanes), a16)
            a16 = a16 * SCALE
            alph[pl.ds(ch * CHUNK + g * L, L)] = a16
            epos = st8 + ch * CHUNK + g * L + lanes
            valid = jnp.logical_and(epos >= st, epos < en)
            return jnp.maximum(mrun, jnp.where(valid, a16, NEG))

        return lax.fori_loop(0, GPC, group, mrun)

    mrun = lax.fori_loop(0, NCH, chunk_a, jnp.full((L,), NEG, jnp.float32))
    mx = _lane_max(mrun, lanes)

    def exloop(t, carry):
        a16 = alph[pl.ds(t * L, L)]
        epos = st8 + t * L + lanes
        valid = jnp.logical_and(epos >= st, epos < en)
        alph[pl.ds(t * L, L)] = jnp.where(valid, jnp.exp(a16 - mx), 0.0)
        return carry

    lax.fori_loop(0, EPTS // L, exloop, 0)

    zv = jnp.zeros((L,), jnp.float32)

    def chunk_b(ch, carry):
        vcp = pltpu.async_copy(v_hbm.at[src1.at[pl.ds(ch * CHUNK, CHUNK)]],
                               krows, sem1)
        vcp.wait()

        def group(g, carry):
            accs = list(carry[0])
            den = carry[1]
            cur = carry[2]
            exv = alph[pl.ds(ch * CHUNK + g * L, L)]
            vd = dst1[pl.ds(ch * CHUNK + g * L, L)]
            for u in range(L):
                e = g * L + u
                d = vd[u]
                ne = d != cur
                cur = jnp.where(ne, d, cur)
                exu = exv[u]
                for j in range(D // L):
                    prev = jnp.where(ne, zv, accs[j])
                    accs[j] = prev + krows[e, pl.ds(j * L, L)] * exu
                den = jnp.where(ne, zv, den) + exu
                inr = jnp.logical_and(cur >= node0, cur < node1)
                slot = jnp.where(inr, cur - node0, NPT)
                base = slot * D
                for j in range(D // L):
                    outb[pl.ds(base + j * L, L)] = accs[j]
                denb[pl.ds(slot * L, L)] = den
            return tuple(accs), den, cur

        return lax.fori_loop(0, GPC, group, carry)

    init = (tuple(jnp.zeros((L,), jnp.float32) for _ in range(D // L)),
            jnp.zeros((L,), jnp.float32), jnp.int32(-1))
    lax.fori_loop(0, NCH, chunk_b, init)

    pltpu.sync_copy(outb.at[pl.ds(0, NPT * D)],
                    agg_hbm.at[pl.ds(wid * NPT * D, NPT * D)])
    pltpu.sync_copy(denb.at[pl.ds(0, NPT * L)],
                    den_hbm.at[pl.ds(wid * NPT * L, NPT * L)])


_edge_call = functools.partial(
    pl.kernel,
    out_type=[
        jax.ShapeDtypeStruct((NP * D,), jnp.float32),
        jax.ShapeDtypeStruct((NP * L,), jnp.float32),
    ],
    mesh=plsc.VectorSubcoreMesh(core_axis_name="c", subcore_axis_name="s"),
    scratch_types=[
        pltpu.VMEM((EPTS,), jnp.int32),
        pltpu.VMEM((EPTS,), jnp.int32),
        pltpu.VMEM((EPTS,), jnp.float32),
        pltpu.VMEM((CHUNK, D), jnp.float32),
        pltpu.VMEM((CHUNK, D), jnp.float32),
        pltpu.VMEM((L,), jnp.int32),
        pltpu.VMEM(((NPT + 1) * D,), jnp.float32),
        pltpu.VMEM(((NPT + 1) * L,), jnp.float32),
        pltpu.SemaphoreType.DMA,
        pltpu.SemaphoreType.DMA,
    ],
    name="edge_attn",
)(_edge_body)


def _qkvs_body(x_ref, w_ref, b_ref, q_ref, k_ref, v_ref, s_ref):
    acc = jnp.dot(x_ref[...], w_ref[...],
                  preferred_element_type=jnp.float32) + b_ref[...]
    q_ref[...] = acc[:, 0 * D:1 * D]
    k_ref[...] = acc[:, 1 * D:2 * D]
    v_ref[...] = acc[:, 2 * D:3 * D]
    s_ref[...] = acc[:, 3 * D:4 * D]


def _qkvs(x, wcat, bcat):
    return pl.pallas_call(
        _qkvs_body,
        grid=(N // BLK,),
        in_specs=[
            pl.BlockSpec((BLK, D), lambda i: (i, 0)),
            pl.BlockSpec((D, 4 * D), lambda i: (0, 0)),
            pl.BlockSpec((1, 4 * D), lambda i: (0, 0)),
        ],
        out_specs=[pl.BlockSpec((BLK, D), lambda i: (i, 0))] * 4,
        out_shape=[jax.ShapeDtypeStruct((N, D), jnp.float32)] * 4,
    )(x, wcat, bcat)


def _merge(agg_ref, den_ref, skip_ref):
    return agg_ref[...] / (den_ref[:, 0:1] + 1e-16) + skip_ref[...]


def _comb_mm_body(agg_ref, den_ref, skip_ref, w_ref, b_ref,
                  q_ref, k_ref, v_ref, s_ref):
    h = jnp.maximum(_merge(agg_ref, den_ref, skip_ref), 0.0)
    acc = jnp.dot(h, w_ref[...],
                  preferred_element_type=jnp.float32) + b_ref[...]
    q_ref[...] = acc[:, 0 * D:1 * D]
    k_ref[...] = acc[:, 1 * D:2 * D]
    v_ref[...] = acc[:, 2 * D:3 * D]
    s_ref[...] = acc[:, 3 * D:4 * D]


def _comb_mm(agg, den, skip, wcat, bcat):
    return pl.pallas_call(
        _comb_mm_body,
        grid=(N // BLK,),
        in_specs=[
            pl.BlockSpec((BLK, D), lambda i: (i, 0)),
            pl.BlockSpec((BLK, L), lambda i: (i, 0)),
            pl.BlockSpec((BLK, D), lambda i: (i, 0)),
            pl.BlockSpec((D, 4 * D), lambda i: (0, 0)),
            pl.BlockSpec((1, 4 * D), lambda i: (0, 0)),
        ],
        out_specs=[pl.BlockSpec((BLK, D), lambda i: (i, 0))] * 4,
        out_shape=[jax.ShapeDtypeStruct((N, D), jnp.float32)] * 4,
    )(agg, den, skip, wcat, bcat)


def _final_body(agg_ref, den_ref, skip_ref, o_ref):
    o_ref[...] = jax.nn.sigmoid(_merge(agg_ref, den_ref, skip_ref))


def _final(agg, den, skip):
    return pl.pallas_call(
        _final_body,
        grid=(N // BLK,),
        in_specs=[
            pl.BlockSpec((BLK, D), lambda i: (i, 0)),
            pl.BlockSpec((BLK, L), lambda i: (i, 0)),
            pl.BlockSpec((BLK, D), lambda i: (i, 0)),
        ],
        out_specs=pl.BlockSpec((BLK, D), lambda i: (i, 0)),
        out_shape=jax.ShapeDtypeStruct((N, D), jnp.float32),
    )(agg, den, skip)


def kernel(x, edge_index, Wq0, bq0, Wk0, bk0, Wv0, bv0, Ws0, bs0,
           Wq1, bq1, Wk1, bk1, Wv1, bv1, Ws1, bs1,
           Wq2, bq2, Wk2, bk2, Wv2, bv2, Ws2, bs2):
    src = edge_index[0]
    dst = edge_index[1]
    order = jnp.argsort(dst)
    srcs = jnp.take(src, order)
    dsts = jnp.take(dst, order)
    pad = jnp.zeros((EPTS,), jnp.int32)
    srcp = jnp.concatenate([srcs, pad])
    dstp = jnp.concatenate([dsts, pad])
    starts = jnp.searchsorted(
        dsts, jnp.arange(NW, dtype=jnp.int32) * NPT).astype(jnp.int32)
    ends = jnp.concatenate([starts[1:], jnp.array([E], jnp.int32)])
    meta = (jnp.zeros((NW, L), jnp.int32)
            .at[:, 0].set(starts).at[:, 1].set(ends))
    zagg = jnp.zeros((NPT * D,), jnp.float32)
    zden = jnp.zeros((NPT * L,), jnp.float32)

    def cat(wq, bq, wk, bk, wv, bv, ws, bs):
        w = jnp.concatenate([wq, wk, wv, ws], axis=1)
        b = jnp.concatenate([bq, bk, bv, bs]).reshape(1, 4 * D)
        return w, b

    w0, b0 = cat(Wq0, bq0, Wk0, bk0, Wv0, bv0, Ws0, bs0)
    w1, b1 = cat(Wq1, bq1, Wk1, bk1, Wv1, bv1, Ws1, bs1)
    w2, b2 = cat(Wq2, bq2, Wk2, bk2, Wv2, bv2, Ws2, bs2)

    def edge(q, k, v):
        agg, den = _edge_call(q, k, v, srcp, dstp, meta, zagg, zden)
        return agg.reshape(NP, D)[:N], den.reshape(NP, L)[:N]

    q, k, v, sk = _qkvs(x, w0, b0)
    agg, den = edge(q, k, v)
    q, k, v, sk = _comb_mm(agg, den, sk, w1, b1)
    agg, den = edge(q, k, v)
    q, k, v, sk = _comb_mm(agg, den, sk, w2, b2)
    agg, den = edge(q, k, v)
    return _final(agg, den, sk)


# speedup vs baseline: 5.6112x; 1.1916x over previous
import functools
import math

import jax
import jax.numpy as jnp
from jax import lax
from jax.experimental import pallas as pl
from jax.experimental.pallas import tpu as pltpu
from jax.experimental.pallas import tpu_sc as plsc

N = 10000
E = 320000
D = 128
NC = 2
NS = 16
L = 16
NW = NC * NS
NP = 10240
NPT = NP // NW
EPTS = 10880
CHUNK = 80
NCH = EPTS // CHUNK
GPC = CHUNK // L
SCALE = 1.0 / math.sqrt(float(D))
BLK = 1000
NEG = -3.0e38


def _lane_sum(v, lanes):
    for sh in (8, 4, 2, 1):
        v = v + v.at[lanes ^ sh].get(mode="promise_in_bounds")
    return v


def _lane_max(v, lanes):
    for sh in (8, 4, 2, 1):
        v = jnp.maximum(v, v.at[lanes ^ sh].get(mode="promise_in_bounds"))
    return v


def _edge_body(q_hbm, k_hbm, v_hbm, srcp_hbm, dstp_hbm, meta_hbm,
               zagg_hbm, zden_hbm, agg_hbm, den_hbm,
               src1, dst1, alph, r0, r1, r2, r3, metav, outb, denb,
               sq0, sk0, sq1, sk1):
    c = lax.axis_index("c")
    s = lax.axis_index("s")
    wid = c * NS + s
    node0 = wid * NPT
    node1 = node0 + NPT

    pltpu.sync_copy(meta_hbm.at[wid], metav)
    mv = metav[pl.ds(0, L)]
    st = mv[0]
    en = mv[1]
    st8 = (st // 8) * 8

    pltpu.sync_copy(srcp_hbm.at[pl.ds(st8, EPTS)], src1.at[pl.ds(0, EPTS)])
    pltpu.sync_copy(dstp_hbm.at[pl.ds(st8, EPTS)], dst1.at[pl.ds(0, EPTS)])
    pltpu.sync_copy(zagg_hbm, outb.at[pl.ds(0, NPT * D)])
    pltpu.sync_copy(zden_hbm, denb.at[pl.ds(0, NPT * L)])
    zi = jnp.zeros((L,), jnp.int32)
    for t_ in range(CHUNK // L):
        src1[pl.ds(EPTS + t_ * L, L)] = zi
        dst1[pl.ds(EPTS + t_ * L, L)] = zi

    lanes = lax.iota(jnp.int32, L)

    def gather(table, idx1, ch, rows, sem):
        pltpu.async_copy(table.at[idx1.at[pl.ds(ch * CHUNK, CHUNK)]],
                         rows, sem)

    def dots(ch, qrows, krows, mrun):
        def group(g, mrun):
            a16 = jnp.zeros((L,), jnp.float32)
            for u in range(L):
                e = g * L + u
                acc = jnp.zeros((L,), jnp.float32)
                for j in range(D // L):
                    acc = acc + (qrows[e, pl.ds(j * L, L)] *
                                 krows[e, pl.ds(j * L, L)])
                a16 = jnp.where(lanes == u, _lane_sum(acc, lanes), a16)
            a16 = a16 * SCALE
            alph[pl.ds(ch * CHUNK + g * L, L)] = a16
            epos = st8 + ch * CHUNK + g * L + lanes
            valid = jnp.logical_and(epos >= st, epos < en)
            return jnp.maximum(mrun, jnp.where(valid, a16, NEG))

        return lax.fori_loop(0, GPC, group, mrun)

    gather(q_hbm, dst1, 0, r0, sq0)
    gather(k_hbm, src1, 0, r2, sk0)

    def pair_a(p, mrun):
        ch0 = 2 * p
        gather(q_hbm, dst1, ch0 + 1, r1, sq1)
        gather(k_hbm, src1, ch0 + 1, r3, sk1)
        pltpu.make_async_copy(q_hbm.at[pl.ds(0, CHUNK)], r0, sq0).wait()
        pltpu.make_async_copy(k_hbm.at[pl.ds(0, CHUNK)], r2, sk0).wait()
        mrun = dots(ch0, r0, r2, mrun)
        gather(q_hbm, dst1, ch0 + 2, r0, sq0)
        gather(k_hbm, src1, ch0 + 2, r2, sk0)
        pltpu.make_async_copy(q_hbm.at[pl.ds(0, CHUNK)], r1, sq1).wait()
        pltpu.make_async_copy(k_hbm.at[pl.ds(0, CHUNK)], r3, sk1).wait()
        return dots(ch0 + 1, r1, r3, mrun)

    mrun = lax.fori_loop(0, NCH // 2, pair_a,
                         jnp.full((L,), NEG, jnp.float32))
    pltpu.make_async_copy(q_hbm.at[pl.ds(0, CHUNK)], r0, sq0).wait()
    pltpu.make_async_copy(k_hbm.at[pl.ds(0, CHUNK)], r2, sk0).wait()
    mx = _lane_max(mrun, lanes)

    def exloop(t, carry):
        a16 = alph[pl.ds(t * L, L)]
        epos = st8 + t * L + lanes
        valid = jnp.logical_and(epos >= st, epos < en)
        alph[pl.ds(t * L, L)] = jnp.where(valid, jnp.exp(a16 - mx), 0.0)
        return carry

    lax.fori_loop(0, EPTS // L, exloop, 0)

    zv = jnp.zeros((L,), jnp.float32)

    def bgroup(ch, krows, carry):
        def group(g, carry):
            accs = list(carry[0])
            den = carry[1]
            cur = carry[2]
            exv = alph[pl.ds(ch * CHUNK + g * L, L)]
            vd = dst1[pl.ds(ch * CHUNK + g * L, L)]
            for u in range(L):
                e = g * L + u
                d = vd[u]
                ne = d != cur
                cur = jnp.where(ne, d, cur)
                exu = exv[u]
                for j in range(D // L):
                    prev = jnp.where(ne, zv, accs[j])
                    accs[j] = prev + krows[e, pl.ds(j * L, L)] * exu
                den = jnp.where(ne, zv, den) + exu
                inr = jnp.logical_and(cur >= node0, cur < node1)
                slot = jnp.where(inr, cur - node0, NPT)
                base = slot * D
                for j in range(D // L):
                    outb[pl.ds(base + j * L, L)] = accs[j]
                denb[pl.ds(slot * L, L)] = den
            return tuple(accs), den, cur

        return lax.fori_loop(0, GPC, group, carry)

    gather(v_hbm, src1, 0, r0, sq0)

    def pair_b(p, carry):
        ch0 = 2 * p
        gather(v_hbm, src1, ch0 + 1, r1, sq1)
        pltpu.make_async_copy(v_hbm.at[pl.ds(0, CHUNK)], r0, sq0).wait()
        carry = bgroup(ch0, r0, carry)
        gather(v_hbm, src1, ch0 + 2, r0, sq0)
        pltpu.make_async_copy(v_hbm.at[pl.ds(0, CHUNK)], r1, sq1).wait()
        return bgroup(ch0 + 1, r1, carry)

    init = (tuple(jnp.zeros((L,), jnp.float32) for _ in range(D // L)),
            jnp.zeros((L,), jnp.float32), jnp.int32(-1))
    lax.fori_loop(0, NCH // 2, pair_b, init)
    pltpu.make_async_copy(v_hbm.at[pl.ds(0, CHUNK)], r0, sq0).wait()

    pltpu.sync_copy(outb.at[pl.ds(0, NPT * D)],
                    agg_hbm.at[pl.ds(wid * NPT * D, NPT * D)])
    pltpu.sync_copy(denb.at[pl.ds(0, NPT * L)],
                    den_hbm.at[pl.ds(wid * NPT * L, NPT * L)])


_edge_call = functools.partial(
    pl.kernel,
    out_type=[
        jax.ShapeDtypeStruct((NP * D,), jnp.float32),
        jax.ShapeDtypeStruct((NP * L,), jnp.float32),
    ],
    mesh=plsc.VectorSubcoreMesh(core_axis_name="c", subcore_axis_name="s"),
    scratch_types=[
        pltpu.VMEM((EPTS + CHUNK,), jnp.int32),
        pltpu.VMEM((EPTS + CHUNK,), jnp.int32),
        pltpu.VMEM((EPTS,), jnp.float32),
        pltpu.VMEM((CHUNK, D), jnp.float32),
        pltpu.VMEM((CHUNK, D), jnp.float32),
        pltpu.VMEM((CHUNK, D), jnp.float32),
        pltpu.VMEM((CHUNK, D), jnp.float32),
        pltpu.VMEM((L,), jnp.int32),
        pltpu.VMEM(((NPT + 1) * D,), jnp.float32),
        pltpu.VMEM(((NPT + 1) * L,), jnp.float32),
        pltpu.SemaphoreType.DMA,
        pltpu.SemaphoreType.DMA,
        pltpu.SemaphoreType.DMA,
        pltpu.SemaphoreType.DMA,
    ],
    name="edge_attn",
)(_edge_body)


def _qkvs_body(x_ref, w_ref, b_ref, q_ref, k_ref, v_ref, s_ref):
    acc = jnp.dot(x_ref[...], w_ref[...],
                  preferred_element_type=jnp.float32) + b_ref[...]
    q_ref[...] = acc[:, 0 * D:1 * D]
    k_ref[...] = acc[:, 1 * D:2 * D]
    v_ref[...] = acc[:, 2 * D:3 * D]
    s_ref[...] = acc[:, 3 * D:4 * D]


def _qkvs(x, wcat, bcat):
    return pl.pallas_call(
        _qkvs_body,
        grid=(N // BLK,),
        in_specs=[
            pl.BlockSpec((BLK, D), lambda i: (i, 0)),
            pl.BlockSpec((D, 4 * D), lambda i: (0, 0)),
            pl.BlockSpec((1, 4 * D), lambda i: (0, 0)),
        ],
        out_specs=[pl.BlockSpec((BLK, D), lambda i: (i, 0))] * 4,
        out_shape=[jax.ShapeDtypeStruct((N, D), jnp.float32)] * 4,
    )(x, wcat, bcat)


def _merge(agg_ref, den_ref, skip_ref):
    return agg_ref[...] / (den_ref[:, 0:1] + 1e-16) + skip_ref[...]


def _comb_mm_body(agg_ref, den_ref, skip_ref, w_ref, b_ref,
                  q_ref, k_ref, v_ref, s_ref):
    h = jnp.maximum(_merge(agg_ref, den_ref, skip_ref), 0.0)
    acc = jnp.dot(h, w_ref[...],
                  preferred_element_type=jnp.float32) + b_ref[...]
    q_ref[...] = acc[:, 0 * D:1 * D]
    k_ref[...] = acc[:, 1 * D:2 * D]
    v_ref[...] = acc[:, 2 * D:3 * D]
    s_ref[...] = acc[:, 3 * D:4 * D]


def _comb_mm(agg, den, skip, wcat, bcat):
    return pl.pallas_call(
        _comb_mm_body,
        grid=(N // BLK,),
        in_specs=[
            pl.BlockSpec((BLK, D), lambda i: (i, 0)),
            pl.BlockSpec((BLK, L), lambda i: (i, 0)),
            pl.BlockSpec((BLK, D), lambda i: (i, 0)),
            pl.BlockSpec((D, 4 * D), lambda i: (0, 0)),
            pl.BlockSpec((1, 4 * D), lambda i: (0, 0)),
        ],
        out_specs=[pl.BlockSpec((BLK, D), lambda i: (i, 0))] * 4,
        out_shape=[jax.ShapeDtypeStruct((N, D), jnp.float32)] * 4,
    )(agg, den, skip, wcat, bcat)


def _final_body(agg_ref, den_ref, skip_ref, o_ref):
    o_ref[...] = jax.nn.sigmoid(_merge(agg_ref, den_ref, skip_ref))


def _final(agg, den, skip):
    return pl.pallas_call(
        _final_body,
        grid=(N // BLK,),
        in_specs=[
            pl.BlockSpec((BLK, D), lambda i: (i, 0)),
            pl.BlockSpec((BLK, L), lambda i: (i, 0)),
            pl.BlockSpec((BLK, D), lambda i: (i, 0)),
        ],
        out_specs=pl.BlockSpec((BLK, D), lambda i: (i, 0)),
        out_shape=jax.ShapeDtypeStruct((N, D), jnp.float32),
    )(agg, den, skip)


def kernel(x, edge_index, Wq0, bq0, Wk0, bk0, Wv0, bv0, Ws0, bs0,
           Wq1, bq1, Wk1, bk1, Wv1, bv1, Ws1, bs1,
           Wq2, bq2, Wk2, bk2, Wv2, bv2, Ws2, bs2):
    src = edge_index[0]
    dst = edge_index[1]
    order = jnp.argsort(dst)
    srcs = jnp.take(src, order)
    dsts = jnp.take(dst, order)
    pad = jnp.zeros((EPTS,), jnp.int32)
    srcp = jnp.concatenate([srcs, pad])
    dstp = jnp.concatenate([dsts, pad])
    starts = jnp.searchsorted(
        dsts, jnp.arange(NW, dtype=jnp.int32) * NPT).astype(jnp.int32)
    ends = jnp.concatenate([starts[1:], jnp.array([E], jnp.int32)])
    meta = (jnp.zeros((NW, L), jnp.int32)
            .at[:, 0].set(starts).at[:, 1].set(ends))
    zagg = jnp.zeros((NPT * D,), jnp.float32)
    zden = jnp.zeros((NPT * L,), jnp.float32)

    def cat(wq, bq, wk, bk, wv, bv, ws, bs):
        w = jnp.concatenate([wq, wk, wv, ws], axis=1)
        b = jnp.concatenate([bq, bk, bv, bs]).reshape(1, 4 * D)
        return w, b

    w0, b0 = cat(Wq0, bq0, Wk0, bk0, Wv0, bv0, Ws0, bs0)
    w1, b1 = cat(Wq1, bq1, Wk1, bk1, Wv1, bv1, Ws1, bs1)
    w2, b2 = cat(Wq2, bq2, Wk2, bk2, Wv2, bv2, Ws2, bs2)

    def edge(q, k, v):
        agg, den = _edge_call(q, k, v, srcp, dstp, meta, zagg, zden)
        return agg.reshape(NP, D)[:N], den.reshape(NP, L)[:N]

    q, k, v, sk = _qkvs(x, w0, b0)
    agg, den = edge(q, k, v)
    q, k, v, sk = _comb_mm(agg, den, sk, w1, b1)
    agg, den = edge(q, k, v)
    q, k, v, sk = _comb_mm(agg, den, sk, w2, b2)
    agg, den = edge(q, k, v)
    return _final(agg, den, sk)
